# Initial kernel scaffold; baseline (speedup 1.0000x reference)
import functools

import jax
import jax.numpy as jnp
from jax import lax
from jax.experimental import pallas as pl
from jax.experimental.pallas import tpu as pltpu
from jax.experimental.pallas import tpu_sc as plsc

L = 16


def _mm_body(x_ref, w_ref, o_ref):
    o_ref[...] = jnp.dot(x_ref[...], w_ref[...],
                         preferred_element_type=jnp.float32)


def _matmul(x, w, bm):
    m, k = x.shape
    n = w.shape[1]
    return pl.pallas_call(
        _mm_body,
        grid=(m // bm,),
        in_specs=[
            pl.BlockSpec((bm, k), lambda i: (i, 0)),
            pl.BlockSpec((k, n), lambda i: (0, 0)),
        ],
        out_specs=pl.BlockSpec((bm, n), lambda i: (i, 0)),
        out_shape=jax.ShapeDtypeStruct((m, n), jnp.float32),
    )(x, w)


def _out_body(agg_ref, x_ref, w_ref, b_ref, o_ref):
    r = jnp.dot(x_ref[...], w_ref[...], preferred_element_type=jnp.float32)
    v = agg_ref[...] + r + b_ref[...]
    o_ref[...] = jax.nn.gelu(v, approximate=False)


def _epilogue(agg, x, w, b, bm):
    m, k = x.shape
    n = w.shape[1]
    return pl.pallas_call(
        _out_body,
        grid=(m // bm,),
        in_specs=[
            pl.BlockSpec((bm, n), lambda i: (i, 0)),
            pl.BlockSpec((bm, k), lambda i: (i, 0)),
            pl.BlockSpec((k, n), lambda i: (0, 0)),
            pl.BlockSpec((1, n), lambda i: (0, 0)),
        ],
        out_specs=pl.BlockSpec((bm, n), lambda i: (i, 0)),
        out_shape=jax.ShapeDtypeStruct((m, n), jnp.float32),
    )(agg, x, w, b)


def _rsqrt16(x):
    i = plsc.bitcast(x, jnp.int32)
    i = jnp.int32(0x5F3759DF) - lax.shift_right_logical(i, 1)
    y = plsc.bitcast(i, jnp.float32)
    half = x * jnp.float32(0.5)
    for _ in range(3):
        y = y * (jnp.float32(1.5) - half * y * y)
    return y


def _make_scatter(N, T, E, Cout, CH, K):
    NT = N * T
    EW = E // 16
    NG = EW // L
    NCH = -(-N // CH)
    SUB = CH // 16
    KB = 8 * K
    RPT = (T * CH) // 16

    mesh = plsc.VectorSubcoreMesh(core_axis_name="c", subcore_axis_name="s")

    @functools.partial(
        pl.kernel,
        out_type=jax.ShapeDtypeStruct((NT, Cout), jnp.float32),
        mesh=mesh,
        scratch_types=[
            pltpu.VMEM((EW,), jnp.int32),
            pltpu.VMEM((EW,), jnp.int32),
            pltpu.VMEM((EW,), jnp.float32),
            pltpu.VMEM((N,), jnp.float32),
            pltpu.VMEM((EW + L,), jnp.int32),
            pltpu.VMEM((EW + L,), jnp.int32),
            pltpu.VMEM((EW + L,), jnp.float32),
            pltpu.VMEM((KB,), jnp.int32),
            pltpu.VMEM((KB,), jnp.int32),
            pltpu.VMEM((KB, Cout), jnp.float32),
            pltpu.VMEM_SHARED((N,), jnp.float32),
            pltpu.VMEM_SHARED((T * CH, Cout), jnp.float32),
            pltpu.SemaphoreType.DMA,
        ],
    )
    def scatter_kernel(row_hbm, col_hbm, ew_hbm, h_hbm, agg_hbm,
                       row_v, col_v, ew_v, dinv_v, mrow, mcol, mnorm,
                       gidx, didx, staging, deg_sh, accum, sem):
        cid = lax.axis_index("c")
        sid = lax.axis_index("s")
        zero16 = jnp.zeros((L,), jnp.float32)
        zero16i = jnp.zeros((L,), jnp.int32)

        base = sid * EW
        pltpu.sync_copy(row_hbm.at[pl.ds(base, EW)], row_v)
        pltpu.sync_copy(col_hbm.at[pl.ds(base, EW)], col_v)
        pltpu.sync_copy(ew_hbm.at[pl.ds(base, EW)], ew_v)

        def _zrow(r, _):
            def _zg(g, _):
                staging[r, pl.ds(g * L, L)] = zero16
                return 0
            lax.fori_loop(0, Cout // L, _zg, 0)
            return 0
        lax.fori_loop(0, KB, _zrow, 0)

        @pl.when(sid == 0)
        def _():
            nfull = N // Cout
            for z in range(nfull):
                pltpu.sync_copy(staging.at[0], deg_sh.at[pl.ds(z * Cout, Cout)])
            rem = N - nfull * Cout
            if rem:
                pltpu.sync_copy(staging.at[0].at[pl.ds(0, rem)],
                                deg_sh.at[pl.ds(nfull * Cout, rem)])
        plsc.subcore_barrier()
        pltpu.sync_copy(ew_v, deg_sh.at[col_v], add=True)
        plsc.subcore_barrier()

        pltpu.sync_copy(deg_sh, dinv_v)

        def _dinv_body(g, _):
            d = dinv_v[pl.ds(g * L, L)]
            pos = d > jnp.float32(0.0)
            safe = jnp.where(pos, d, jnp.float32(1.0))
            r = _rsqrt16(safe)
            dinv_v[pl.ds(g * L, L)] = jnp.where(pos, r, jnp.float32(0.0))
            return 0
        lax.fori_loop(0, N // L, _dinv_body, 0)

        nch_for_me = jnp.where(cid == 0, (NCH + 1) // 2, NCH // 2)

        def _chunk_body(ci, _):
            c = ci * 2 + cid
            lo = jnp.minimum(c * CH, N - CH)
            hi = lo + CH

            nzc = RPT // KB
            def _zcp(z, _):
                pltpu.sync_copy(
                    staging, accum.at[pl.ds(sid * RPT + z * KB, KB)])
                return 0
            lax.fori_loop(0, nzc, _zcp, 0)
            rem = RPT - nzc * KB
            if rem:
                pltpu.sync_copy(
                    staging.at[pl.ds(0, rem)],
                    accum.at[pl.ds(sid * RPT + nzc * KB, rem)])
            plsc.subcore_barrier()

            def _scan_body(g, cnt):
                c16 = col_v[pl.ds(g * L, L)]
                r16 = row_v[pl.ds(g * L, L)]
                w16 = ew_v[pl.ds(g * L, L)]
                m = (c16 >= lo) & (c16 < hi)
                dr = plsc.load_gather(dinv_v, [r16])
                dc = plsc.load_gather(dinv_v, [c16])
                nrm = dr * w16 * dc
                plsc.store_compressed(mrow.at[pl.ds(cnt, L)], r16, m)
                plsc.store_compressed(mcol.at[pl.ds(cnt, L)], c16 - lo, m)
                plsc.store_compressed(mnorm.at[pl.ds(cnt, L)], nrm, m)
                pc = plsc.all_reduce_population_count(m)
                return cnt + pc[0]
            cnt = lax.fori_loop(0, NG, _scan_body, jnp.int32(0))

            mrow[pl.ds(cnt, L)] = zero16i
            mcol[pl.ds(cnt, L)] = zero16i
            mnorm[pl.ds(cnt, L)] = zero16
            nbatch = (cnt + K - 1) // K

            def _batch_body(b, _):
                e0 = b * K
                for jj in range(K // L):
                    r16 = mrow[pl.ds(e0 + jj * L, L)]
                    c16 = mcol[pl.ds(e0 + jj * L, L)]
                    for t in range(T):
                        gidx[pl.ds(t * K + jj * L, L)] = r16 + t * N
                        didx[pl.ds(t * K + jj * L, L)] = c16 + t * CH
                pltpu.async_copy(h_hbm.at[gidx], staging, sem).wait()

                def _scale_body(mi, _):
                    e = e0 + jnp.bitwise_and(mi, K - 1)
                    ns = plsc.load_gather(mnorm, [jnp.broadcast_to(e, (L,))])
                    for g in range(Cout // L):
                        v = staging[mi, pl.ds(g * L, L)]
                        staging[mi, pl.ds(g * L, L)] = v * ns
                    return 0
                lax.fori_loop(0, KB, _scale_body, 0)
                pltpu.sync_copy(staging, accum.at[didx], add=True)
                return 0
            lax.fori_loop(0, nbatch, _batch_body, 0)

            lax.fori_loop(0, KB, _zrow, 0)
            plsc.subcore_barrier()

            for t in range(T):
                pltpu.sync_copy(
                    accum.at[pl.ds(t * CH + sid * SUB, SUB)],
                    agg_hbm.at[pl.ds(t * N + lo + sid * SUB, SUB)])
            plsc.subcore_barrier()
            return 0

        lax.fori_loop(0, nch_for_me, _chunk_body, 0)

    return scatter_kernel


def kernel(x, edge_index, edge_attr, W_init, W_root, bias):
    N, T, C = x.shape
    E = edge_attr.shape[0]
    NT = N * T
    Cout = W_init.shape[1]
    x_flat = x.reshape(NT, C)
    row = edge_index[0]
    col = edge_index[1]

    h = _matmul(x_flat, W_init, bm=800)

    scatter = _make_scatter(N, T, E, Cout, CH=992, K=16)
    agg = scatter(row, col, edge_attr, h)

    out = _epilogue(agg, x_flat, W_root, bias.reshape(1, Cout), bm=800)
    return out.reshape(N, T, Cout)


# speedup vs baseline: 4.1669x; 4.1669x over previous
import functools

import jax
import jax.numpy as jnp
from jax import lax
from jax.experimental import pallas as pl
from jax.experimental.pallas import tpu as pltpu
from jax.experimental.pallas import tpu_sc as plsc

L = 16


def _mm_body(x_ref, w_ref, o_ref):
    o_ref[...] = jnp.dot(x_ref[...], w_ref[...],
                         preferred_element_type=jnp.float32)


def _matmul(x, w, bm):
    m, k = x.shape
    n = w.shape[1]
    return pl.pallas_call(
        _mm_body,
        grid=(m // bm,),
        in_specs=[
            pl.BlockSpec((bm, k), lambda i: (i, 0)),
            pl.BlockSpec((k, n), lambda i: (0, 0)),
        ],
        out_specs=pl.BlockSpec((bm, n), lambda i: (i, 0)),
        out_shape=jax.ShapeDtypeStruct((m, n), jnp.float32),
    )(x, w)


def _out_body(agg_ref, x_ref, w_ref, b_ref, o_ref):
    r = jnp.dot(x_ref[...], w_ref[...], preferred_element_type=jnp.float32)
    v = agg_ref[...] + r + b_ref[...]
    o_ref[...] = v * 0.5 * (1.0 + lax.erf(v * (2.0 ** -0.5)))


def _epilogue(agg, x, w, b, bm):
    m, k = x.shape
    n = w.shape[1]
    return pl.pallas_call(
        _out_body,
        grid=(m // bm,),
        in_specs=[
            pl.BlockSpec((bm, n), lambda i: (i, 0)),
            pl.BlockSpec((bm, k), lambda i: (i, 0)),
            pl.BlockSpec((k, n), lambda i: (0, 0)),
            pl.BlockSpec((1, n), lambda i: (0, 0)),
        ],
        out_specs=pl.BlockSpec((bm, n), lambda i: (i, 0)),
        out_shape=jax.ShapeDtypeStruct((m, n), jnp.float32),
    )(agg, x, w, b)


def _make_deg(N, E, NP):
    EW2 = E // 32
    mesh = plsc.VectorSubcoreMesh(core_axis_name="c", subcore_axis_name="s")

    @functools.partial(
        pl.kernel,
        out_type=jax.ShapeDtypeStruct((2 * NP,), jnp.float32),
        mesh=mesh,
        compiler_params=pltpu.CompilerParams(needs_layout_passes=False),
        scratch_types=[
            pltpu.VMEM((EW2,), jnp.int32),
            pltpu.VMEM((EW2,), jnp.float32),
            pltpu.VMEM((1280,), jnp.float32),
            pltpu.VMEM_SHARED((NP,), jnp.float32),
        ],
    )
    def deg_kernel(col_hbm, ew_hbm, out_hbm, colw, eww, zbuf, deg_sh):
        cid = lax.axis_index("c")
        sid = lax.axis_index("s")
        base = (cid * 16 + sid) * EW2
        pltpu.sync_copy(col_hbm.at[pl.ds(base, EW2)], colw)
        pltpu.sync_copy(ew_hbm.at[pl.ds(base, EW2)], eww)
        zero16 = jnp.zeros((L,), jnp.float32)

        def _z(g, _):
            zbuf[pl.ds(g * L, L)] = zero16
            return 0
        lax.fori_loop(0, 1280 // L, _z, 0)

        @pl.when(sid == 0)
        def _():
            for z in range(NP // 1280):
                pltpu.sync_copy(zbuf, deg_sh.at[pl.ds(z * 1280, 1280)])
        plsc.subcore_barrier()
        pltpu.sync_copy(eww, deg_sh.at[colw], add=True)
        plsc.subcore_barrier()

        @pl.when(sid == 0)
        def _():
            pltpu.sync_copy(deg_sh, out_hbm.at[pl.ds(cid * NP, NP)])

    return deg_kernel


def _make_norm(N, E, NP):
    EW2 = E // 32
    mesh = plsc.VectorSubcoreMesh(core_axis_name="c", subcore_axis_name="s")

    @functools.partial(
        pl.kernel,
        out_type=jax.ShapeDtypeStruct((E,), jnp.float32),
        mesh=mesh,
        compiler_params=pltpu.CompilerParams(needs_layout_passes=False),
        scratch_types=[
            pltpu.VMEM((EW2 + L,), jnp.int32),
            pltpu.VMEM((EW2 + L,), jnp.int32),
            pltpu.VMEM((EW2 + L,), jnp.float32),
            pltpu.VMEM((NP,), jnp.float32),
        ],
    )
    def norm_kernel(row_hbm, col_hbm, ew_hbm, dinv_hbm, out_hbm,
                    roww, colw, eww, dinv_v):
        cid = lax.axis_index("c")
        sid = lax.axis_index("s")
        base = (cid * 16 + sid) * EW2
        pltpu.sync_copy(row_hbm.at[pl.ds(base, EW2)], roww.at[pl.ds(0, EW2)])
        pltpu.sync_copy(col_hbm.at[pl.ds(base, EW2)], colw.at[pl.ds(0, EW2)])
        pltpu.sync_copy(ew_hbm.at[pl.ds(base, EW2)], eww.at[pl.ds(0, EW2)])
        pltpu.sync_copy(dinv_hbm, dinv_v)
        roww[pl.ds(EW2, L)] = jnp.zeros((L,), jnp.int32)
        colw[pl.ds(EW2, L)] = jnp.zeros((L,), jnp.int32)

        def _body(g, _):
            r16 = roww[pl.ds(g * L, L)]
            c16 = colw[pl.ds(g * L, L)]
            w16 = eww[pl.ds(g * L, L)]
            dr = plsc.load_gather(dinv_v, [r16])
            dc = plsc.load_gather(dinv_v, [c16])
            eww[pl.ds(g * L, L)] = dr * w16 * dc
            return 0
        lax.fori_loop(0, -(-EW2 // L), _body, 0)
        pltpu.sync_copy(eww.at[pl.ds(0, EW2)], out_hbm.at[pl.ds(base, EW2)])

    return norm_kernel


def _dinv_body_tc(degp_ref, o_ref):
    d = degp_ref[0] + degp_ref[1]
    pos = d > 0.0
    safe = jnp.where(pos, d, 1.0)
    o_ref[...] = jnp.where(pos, lax.rsqrt(safe), 0.0)


def _dinv_tc(degp):
    _, r, c = degp.shape
    return pl.pallas_call(
        _dinv_body_tc,
        out_shape=jax.ShapeDtypeStruct((r, c), jnp.float32),
    )(degp)


def _make_scatter(N, T, E, Cout, R, W, K):
    NT = N * T
    NW = 32
    NPASS = -(-N // (NW * R))
    NWIN = E // W
    WG = W // L
    KB = T * K

    mesh = plsc.VectorSubcoreMesh(core_axis_name="c", subcore_axis_name="s")

    @functools.partial(
        pl.kernel,
        out_type=jax.ShapeDtypeStruct((NT, Cout), jnp.float32),
        mesh=mesh,
        compiler_params=pltpu.CompilerParams(needs_layout_passes=False),
        scratch_types=[
            pltpu.VMEM((T * R, Cout), jnp.float32),
            pltpu.VMEM((W,), jnp.int32),
            pltpu.VMEM((W + L,), jnp.int32),
            pltpu.VMEM((W + L,), jnp.int32),
            pltpu.VMEM((KB, Cout), jnp.float32),
            pltpu.VMEM((KB,), jnp.int32),
            pltpu.VMEM((K,), jnp.int32),
            pltpu.VMEM((K,), jnp.float32),
            pltpu.SemaphoreType.DMA,
        ],
    )
    def scatter_kernel(row_hbm, col_hbm, norm_hbm, h_hbm, agg_hbm,
                       acc, win, mpos, mlcol, staging, gidx, rbuf, nbuf, sem):
        cid = lax.axis_index("c")
        sid = lax.axis_index("s")
        wid = cid * 16 + sid
        zero16 = jnp.zeros((L,), jnp.float32)
        iota16 = lax.iota(jnp.int32, L)

        def _pass_body(p, _):
            lo = pl.multiple_of(
                jnp.minimum((p * NW + wid) * R, N - R), 8)
            hi = lo + R

            def _zrow(r, _):
                def _zg(g, _):
                    acc[r, pl.ds(g * L, L)] = zero16
                    return 0
                lax.fori_loop(0, Cout // L, _zg, 0)
                return 0
            lax.fori_loop(0, T * R, _zrow, 0)

            def _win_body(w, _):
                pltpu.sync_copy(col_hbm.at[pl.ds(w * W, W)], win)

                def _scan_body(g, cnt):
                    c16 = win[pl.ds(g * L, L)]
                    m = (c16 >= lo) & (c16 < hi)
                    plsc.store_compressed(mpos.at[pl.ds(cnt, L)],
                                          w * W + g * L + iota16, mask=m)
                    plsc.store_compressed(mlcol.at[pl.ds(cnt, L)],
                                          c16 - lo, mask=m)
                    pc = plsc.all_reduce_population_count(m)
                    return cnt + pc[0]
                cnt = lax.fori_loop(0, WG, _scan_body, jnp.int32(0))

                mpos[pl.ds(cnt, L)] = jnp.zeros((L,), jnp.int32)
                mlcol[pl.ds(cnt, L)] = jnp.full((L,), R, jnp.int32)
                nbatch = (cnt + K - 1) // K

                def _batch_body(b, _):
                    e0 = b * K
                    pltpu.async_copy(row_hbm.at[mpos.at[pl.ds(e0, K)]],
                                     rbuf, sem).wait()
                    pltpu.async_copy(norm_hbm.at[mpos.at[pl.ds(e0, K)]],
                                     nbuf, sem).wait()
                    r16 = rbuf[pl.ds(0, L)]
                    for t in range(T):
                        gidx[pl.ds(t * K, L)] = r16 + t * N
                    pltpu.async_copy(h_hbm.at[gidx], staging, sem).wait()

                    def _acc_body(mi, _):
                        j = jnp.bitwise_and(mi, K - 1)
                        t = lax.shift_right_logical(mi, 4)
                        lcolv = plsc.load_gather(
                            mlcol, [jnp.broadcast_to(e0 + j, (L,))])
                        lcol = lcolv[0]
                        valid = lcol < R
                        lc = jnp.minimum(lcol, R - 1)
                        nsv = plsc.load_gather(
                            nbuf, [jnp.broadcast_to(j, (L,))])
                        nsv = jnp.where(valid, nsv, zero16)
                        dst = t * R + lc
                        for g in range(Cout // L):
                            v = staging[mi, pl.ds(g * L, L)] * nsv
                            plsc.addupdate(acc.at[dst, pl.ds(g * L, L)], v)
                        return 0
                    lax.fori_loop(0, KB, _acc_body, 0)
                    return 0
                lax.fori_loop(0, nbatch, _batch_body, 0)
                return 0
            lax.fori_loop(0, NWIN, _win_body, 0)

            for t in range(T):
                pltpu.sync_copy(acc.at[pl.ds(t * R, R)],
                                agg_hbm.at[pl.ds(t * N + lo, R)])
            return 0
        lax.fori_loop(0, NPASS, _pass_body, 0)

    return scatter_kernel


def kernel(x, edge_index, edge_attr, W_init, W_root, bias):
    N, T, C = x.shape
    E = edge_attr.shape[0]
    NT = N * T
    Cout = W_init.shape[1]
    x_flat = x.reshape(NT, C)
    row = edge_index[0]
    col = edge_index[1]

    NP = 10240
    deg = _make_deg(N, E, NP)(col, edge_attr)
    dinv = _dinv_tc(deg.reshape(2, NP // 128, 128)).reshape(NP)
    norm = _make_norm(N, E, NP)(row, col, edge_attr, dinv)

    h = _matmul(x_flat, W_init, bm=800)

    scatter = _make_scatter(N, T, E, Cout, R=40, W=3200, K=16)
    agg = scatter(row, col, norm, h)

    out = _epilogue(agg, x_flat, W_root, bias.reshape(1, Cout), bm=800)
    return out.reshape(N, T, Cout)


# speedup vs baseline: 4.5560x; 1.0934x over previous
import functools

import jax
import jax.numpy as jnp
from jax import lax
from jax.experimental import pallas as pl
from jax.experimental.pallas import tpu as pltpu
from jax.experimental.pallas import tpu_sc as plsc

L = 16


def _mm_body(x_ref, w_ref, o_ref):
    o_ref[...] = jnp.dot(x_ref[...], w_ref[...],
                         preferred_element_type=jnp.float32)


def _matmul(x, w, bm):
    m, k = x.shape
    n = w.shape[1]
    return pl.pallas_call(
        _mm_body,
        grid=(m // bm,),
        in_specs=[
            pl.BlockSpec((bm, k), lambda i: (i, 0)),
            pl.BlockSpec((k, n), lambda i: (0, 0)),
        ],
        out_specs=pl.BlockSpec((bm, n), lambda i: (i, 0)),
        out_shape=jax.ShapeDtypeStruct((m, n), jnp.float32),
    )(x, w)


def _out_body(agg_ref, x_ref, w_ref, b_ref, o_ref):
    r = jnp.dot(x_ref[...], w_ref[...], preferred_element_type=jnp.float32)
    v = agg_ref[...] + r + b_ref[...]
    o_ref[...] = v * 0.5 * (1.0 + lax.erf(v * (2.0 ** -0.5)))


def _epilogue(agg, x, w, b, bm):
    m, k = x.shape
    n = w.shape[1]
    return pl.pallas_call(
        _out_body,
        grid=(m // bm,),
        in_specs=[
            pl.BlockSpec((bm, n), lambda i: (i, 0)),
            pl.BlockSpec((bm, k), lambda i: (i, 0)),
            pl.BlockSpec((k, n), lambda i: (0, 0)),
            pl.BlockSpec((1, n), lambda i: (0, 0)),
        ],
        out_specs=pl.BlockSpec((bm, n), lambda i: (i, 0)),
        out_shape=jax.ShapeDtypeStruct((m, n), jnp.float32),
    )(agg, x, w, b)


def _make_deg(N, E, NP):
    EW2 = E // 32
    mesh = plsc.VectorSubcoreMesh(core_axis_name="c", subcore_axis_name="s")

    @functools.partial(
        pl.kernel,
        out_type=jax.ShapeDtypeStruct((2 * NP,), jnp.float32),
        mesh=mesh,
        compiler_params=pltpu.CompilerParams(needs_layout_passes=False),
        scratch_types=[
            pltpu.VMEM((EW2,), jnp.int32),
            pltpu.VMEM((EW2,), jnp.float32),
            pltpu.VMEM((1280,), jnp.float32),
            pltpu.VMEM_SHARED((NP,), jnp.float32),
        ],
    )
    def deg_kernel(col_hbm, ew_hbm, out_hbm, colw, eww, zbuf, deg_sh):
        cid = lax.axis_index("c")
        sid = lax.axis_index("s")
        base = (cid * 16 + sid) * EW2
        pltpu.sync_copy(col_hbm.at[pl.ds(base, EW2)], colw)
        pltpu.sync_copy(ew_hbm.at[pl.ds(base, EW2)], eww)
        zero16 = jnp.zeros((L,), jnp.float32)

        def _z(g, _):
            zbuf[pl.ds(g * L, L)] = zero16
            return 0
        lax.fori_loop(0, 1280 // L, _z, 0)

        @pl.when(sid == 0)
        def _():
            for z in range(NP // 1280):
                pltpu.sync_copy(zbuf, deg_sh.at[pl.ds(z * 1280, 1280)])
        plsc.subcore_barrier()
        pltpu.sync_copy(eww, deg_sh.at[colw], add=True)
        plsc.subcore_barrier()

        @pl.when(sid == 0)
        def _():
            pltpu.sync_copy(deg_sh, out_hbm.at[pl.ds(cid * NP, NP)])

    return deg_kernel


def _make_norm(N, E, NP):
    EW2 = E // 32
    mesh = plsc.VectorSubcoreMesh(core_axis_name="c", subcore_axis_name="s")

    @functools.partial(
        pl.kernel,
        out_type=jax.ShapeDtypeStruct((E,), jnp.float32),
        mesh=mesh,
        compiler_params=pltpu.CompilerParams(needs_layout_passes=False),
        scratch_types=[
            pltpu.VMEM((EW2 + L,), jnp.int32),
            pltpu.VMEM((EW2 + L,), jnp.int32),
            pltpu.VMEM((EW2 + L,), jnp.float32),
            pltpu.VMEM((NP,), jnp.float32),
        ],
    )
    def norm_kernel(row_hbm, col_hbm, ew_hbm, dinv_hbm, out_hbm,
                    roww, colw, eww, dinv_v):
        cid = lax.axis_index("c")
        sid = lax.axis_index("s")
        base = (cid * 16 + sid) * EW2
        pltpu.sync_copy(row_hbm.at[pl.ds(base, EW2)], roww.at[pl.ds(0, EW2)])
        pltpu.sync_copy(col_hbm.at[pl.ds(base, EW2)], colw.at[pl.ds(0, EW2)])
        pltpu.sync_copy(ew_hbm.at[pl.ds(base, EW2)], eww.at[pl.ds(0, EW2)])
        pltpu.sync_copy(dinv_hbm, dinv_v)
        roww[pl.ds(EW2, L)] = jnp.zeros((L,), jnp.int32)
        colw[pl.ds(EW2, L)] = jnp.zeros((L,), jnp.int32)

        def _body(g, _):
            r16 = roww[pl.ds(g * L, L)]
            c16 = colw[pl.ds(g * L, L)]
            w16 = eww[pl.ds(g * L, L)]
            dr = plsc.load_gather(dinv_v, [r16])
            dc = plsc.load_gather(dinv_v, [c16])
            eww[pl.ds(g * L, L)] = dr * w16 * dc
            return 0
        lax.fori_loop(0, -(-EW2 // L), _body, 0)
        pltpu.sync_copy(eww.at[pl.ds(0, EW2)], out_hbm.at[pl.ds(base, EW2)])

    return norm_kernel


def _dinv_body_tc(degp_ref, o_ref):
    d = degp_ref[0] + degp_ref[1]
    pos = d > 0.0
    safe = jnp.where(pos, d, 1.0)
    o_ref[...] = jnp.where(pos, lax.rsqrt(safe), 0.0)


def _dinv_tc(degp):
    _, r, c = degp.shape
    return pl.pallas_call(
        _dinv_body_tc,
        out_shape=jax.ShapeDtypeStruct((r, c), jnp.float32),
    )(degp)


def _make_scatter(N, T, E, Cout, R, W, K):
    NT = N * T
    NW = 32
    NPASS = -(-N // (NW * R))
    NWIN = E // W
    WG = W // L
    KB = T * K
    GC = 256
    MB = -(-(W + L) // GC) * GC

    mesh = plsc.VectorSubcoreMesh(core_axis_name="c", subcore_axis_name="s")

    @functools.partial(
        pl.kernel,
        out_type=jax.ShapeDtypeStruct((NT, Cout), jnp.float32),
        mesh=mesh,
        compiler_params=pltpu.CompilerParams(needs_layout_passes=False),
        scratch_types=[
            pltpu.VMEM((T * R, Cout), jnp.float32),
            pltpu.VMEM((W,), jnp.int32),
            pltpu.VMEM((MB,), jnp.int32),
            pltpu.VMEM((MB,), jnp.int32),
            pltpu.VMEM((MB,), jnp.float32),
            pltpu.VMEM((KB, Cout), jnp.float32),
            pltpu.SemaphoreType.DMA,
            pltpu.SemaphoreType.DMA,
        ],
    )
    def scatter_kernel(row_hbm, col_hbm, norm_hbm, h_hbm, agg_hbm,
                       acc, win, mpos, rbufw, nbufw, staging, sem0, sem1):
        cid = lax.axis_index("c")
        sid = lax.axis_index("s")
        wid = cid * 16 + sid
        zero16 = jnp.zeros((L,), jnp.float32)
        iota16 = lax.iota(jnp.int32, L)

        def _pass_body(p, _):
            lo = pl.multiple_of(
                jnp.minimum((p * NW + wid) * R, N - R), 8)
            hi = lo + R

            def _zrow(r, _):
                def _zg(g, _):
                    acc[r, pl.ds(g * L, L)] = zero16
                    return 0
                lax.fori_loop(0, Cout // L, _zg, 0)
                return 0
            lax.fori_loop(0, T * R, _zrow, 0)

            def _win_body(w, _):
                pltpu.sync_copy(col_hbm.at[pl.ds(w * W, W)], win)

                def _scan_body(g, cnt):
                    c16 = win[pl.ds(g * L, L)]
                    m = (c16 >= lo) & (c16 < hi)
                    plsc.store_compressed(mpos.at[pl.ds(cnt, L)],
                                          g * L + iota16, mask=m)
                    pc = plsc.all_reduce_population_count(m)
                    return cnt + pc[0]
                cnt = lax.fori_loop(0, WG, _scan_body, jnp.int32(0))

                mpos[pl.ds(cnt, L)] = jnp.zeros((L,), jnp.int32)
                nbatch = (cnt + K - 1) // K

                def _bulk(ch, _):
                    off = ch * GC
                    def _g2e(g, _):
                        mp = mpos[pl.ds(off + g * L, L)]
                        mpos[pl.ds(off + g * L, L)] = jnp.minimum(
                            mp, W - 1) + w * W
                        return 0
                    lax.fori_loop(0, GC // L, _g2e, 0)
                    cpa = pltpu.async_copy(
                        row_hbm.at[mpos.at[pl.ds(off, GC)]],
                        rbufw.at[pl.ds(off, GC)], sem0)
                    cpb = pltpu.async_copy(
                        norm_hbm.at[mpos.at[pl.ds(off, GC)]],
                        nbufw.at[pl.ds(off, GC)], sem1)
                    cpa.wait()
                    cpb.wait()
                    return 0
                nchunk = (cnt + L + GC - 1) // GC
                lax.fori_loop(0, nchunk, _bulk, 0)

                def _batch_body(b, _):
                    e0 = b * K
                    r16 = rbufw[pl.ds(e0, L)]
                    cps0 = []
                    cps1 = []
                    for t in range(T):
                        cp = pltpu.async_copy(
                            h_hbm.at[r16 + t * N],
                            staging.at[pl.ds(t * K, K)],
                            sem0 if t < T // 2 else sem1)
                        (cps0 if t < T // 2 else cps1).append(cp)

                    def _edge_range(t0, t1):
                        def _edge(j, _):
                            mpv = plsc.load_gather(
                                mpos, [jnp.broadcast_to(e0 + j, (L,))])
                            lcolv = plsc.load_gather(
                                win, [jnp.minimum(mpv - w * W, W - 1)])
                            lcol = lcolv[0] - lo
                            valid = (e0 + j) < cnt
                            lc = jnp.clip(lcol, 0, R - 1)
                            nsv = plsc.load_gather(
                                nbufw, [jnp.broadcast_to(e0 + j, (L,))])
                            nsv = jnp.where(valid, nsv, zero16)
                            for t in range(t0, t1):
                                for g in range(Cout // L):
                                    v = staging[t * K + j,
                                                pl.ds(g * L, L)] * nsv
                                    plsc.addupdate(
                                        acc.at[t * R + lc, pl.ds(g * L, L)],
                                        v)
                            return 0
                        lax.fori_loop(0, K, _edge, 0)
                    for cp in cps0:
                        cp.wait()
                    _edge_range(0, T // 2)
                    for cp in cps1:
                        cp.wait()
                    _edge_range(T // 2, T)
                    return 0
                lax.fori_loop(0, nbatch, _batch_body, 0)
                return 0
            lax.fori_loop(0, NWIN, _win_body, 0)

            for t in range(T):
                pltpu.sync_copy(acc.at[pl.ds(t * R, R)],
                                agg_hbm.at[pl.ds(t * N + lo, R)])
            return 0
        lax.fori_loop(0, NPASS, _pass_body, 0)

    return scatter_kernel


def kernel(x, edge_index, edge_attr, W_init, W_root, bias):
    N, T, C = x.shape
    E = edge_attr.shape[0]
    NT = N * T
    Cout = W_init.shape[1]
    x_flat = x.reshape(NT, C)
    row = edge_index[0]
    col = edge_index[1]

    NP = 10240
    deg = _make_deg(N, E, NP)(col, edge_attr)
    dinv = _dinv_tc(deg.reshape(2, NP // 128, 128)).reshape(NP)
    norm = _make_norm(N, E, NP)(row, col, edge_attr, dinv)

    h = _matmul(x_flat, W_init, bm=800)

    scatter = _make_scatter(N, T, E, Cout, R=32, W=6400, K=16)
    agg = scatter(row, col, norm, h)

    out = _epilogue(agg, x_flat, W_root, bias.reshape(1, Cout), bm=800)
    return out.reshape(N, T, Cout)


# speedup vs baseline: 133.1783x; 29.2311x over previous
import functools

import jax
import jax.numpy as jnp
from jax import lax
from jax.experimental import pallas as pl
from jax.experimental.pallas import tpu as pltpu
from jax.experimental.pallas import tpu_sc as plsc

L = 16


def _mm_body(x_ref, w_ref, o_ref):
    o_ref[...] = jnp.dot(x_ref[...], w_ref[...],
                         preferred_element_type=jnp.float32)


def _matmul(x, w, bm):
    m, k = x.shape
    n = w.shape[1]
    return pl.pallas_call(
        _mm_body,
        grid=(m // bm,),
        in_specs=[
            pl.BlockSpec((bm, k), lambda i: (i, 0)),
            pl.BlockSpec((k, n), lambda i: (0, 0)),
        ],
        out_specs=pl.BlockSpec((bm, n), lambda i: (i, 0)),
        out_shape=jax.ShapeDtypeStruct((m, n), jnp.float32),
    )(x, w)


def _out_body(agg_ref, x_ref, w_ref, b_ref, o_ref):
    r = jnp.dot(x_ref[...], w_ref[...], preferred_element_type=jnp.float32)
    v = agg_ref[...] + r + b_ref[...]
    o_ref[...] = v * 0.5 * (1.0 + lax.erf(v * (2.0 ** -0.5)))


def _epilogue(agg, x, w, b, bm):
    m, k = x.shape
    n = w.shape[1]
    return pl.pallas_call(
        _out_body,
        grid=(m // bm,),
        in_specs=[
            pl.BlockSpec((bm, n), lambda i: (i, 0)),
            pl.BlockSpec((bm, k), lambda i: (i, 0)),
            pl.BlockSpec((k, n), lambda i: (0, 0)),
            pl.BlockSpec((1, n), lambda i: (0, 0)),
        ],
        out_specs=pl.BlockSpec((bm, n), lambda i: (i, 0)),
        out_shape=jax.ShapeDtypeStruct((m, n), jnp.float32),
    )(agg, x, w, b)


def _make_deg(N, E, NP):
    EW2 = E // 32
    mesh = plsc.VectorSubcoreMesh(core_axis_name="c", subcore_axis_name="s")

    @functools.partial(
        pl.kernel,
        out_type=jax.ShapeDtypeStruct((2 * NP,), jnp.float32),
        mesh=mesh,
        compiler_params=pltpu.CompilerParams(needs_layout_passes=False),
        scratch_types=[
            pltpu.VMEM((EW2,), jnp.int32),
            pltpu.VMEM((EW2,), jnp.float32),
            pltpu.VMEM((1280,), jnp.float32),
            pltpu.VMEM_SHARED((NP,), jnp.float32),
        ],
    )
    def deg_kernel(col_hbm, ew_hbm, out_hbm, colw, eww, zbuf, deg_sh):
        cid = lax.axis_index("c")
        sid = lax.axis_index("s")
        base = (cid * 16 + sid) * EW2
        pltpu.sync_copy(col_hbm.at[pl.ds(base, EW2)], colw)
        pltpu.sync_copy(ew_hbm.at[pl.ds(base, EW2)], eww)
        zero16 = jnp.zeros((L,), jnp.float32)

        def _z(g, _):
            zbuf[pl.ds(g * L, L)] = zero16
            return 0
        lax.fori_loop(0, 1280 // L, _z, 0)

        @pl.when(sid == 0)
        def _():
            for z in range(NP // 1280):
                pltpu.sync_copy(zbuf, deg_sh.at[pl.ds(z * 1280, 1280)])
        plsc.subcore_barrier()
        pltpu.sync_copy(eww, deg_sh.at[colw], add=True)
        plsc.subcore_barrier()

        @pl.when(sid == 0)
        def _():
            pltpu.sync_copy(deg_sh, out_hbm.at[pl.ds(cid * NP, NP)])

    return deg_kernel


def _make_norm(N, E, NP):
    EW2 = E // 32
    mesh = plsc.VectorSubcoreMesh(core_axis_name="c", subcore_axis_name="s")

    @functools.partial(
        pl.kernel,
        out_type=jax.ShapeDtypeStruct((E,), jnp.float32),
        mesh=mesh,
        compiler_params=pltpu.CompilerParams(needs_layout_passes=False),
        scratch_types=[
            pltpu.VMEM((EW2 + L,), jnp.int32),
            pltpu.VMEM((EW2 + L,), jnp.int32),
            pltpu.VMEM((EW2 + L,), jnp.float32),
            pltpu.VMEM((NP,), jnp.float32),
        ],
    )
    def norm_kernel(row_hbm, col_hbm, ew_hbm, dinv_hbm, out_hbm,
                    roww, colw, eww, dinv_v):
        cid = lax.axis_index("c")
        sid = lax.axis_index("s")
        base = (cid * 16 + sid) * EW2
        pltpu.sync_copy(row_hbm.at[pl.ds(base, EW2)], roww.at[pl.ds(0, EW2)])
        pltpu.sync_copy(col_hbm.at[pl.ds(base, EW2)], colw.at[pl.ds(0, EW2)])
        pltpu.sync_copy(ew_hbm.at[pl.ds(base, EW2)], eww.at[pl.ds(0, EW2)])
        pltpu.sync_copy(dinv_hbm, dinv_v)
        roww[pl.ds(EW2, L)] = jnp.zeros((L,), jnp.int32)
        colw[pl.ds(EW2, L)] = jnp.zeros((L,), jnp.int32)

        def _body(g, _):
            r16 = roww[pl.ds(g * L, L)]
            c16 = colw[pl.ds(g * L, L)]
            w16 = eww[pl.ds(g * L, L)]
            dr = plsc.load_gather(dinv_v, [r16])
            dc = plsc.load_gather(dinv_v, [c16])
            eww[pl.ds(g * L, L)] = dr * w16 * dc
            return 0
        lax.fori_loop(0, -(-EW2 // L), _body, 0)
        pltpu.sync_copy(eww.at[pl.ds(0, EW2)], out_hbm.at[pl.ds(base, EW2)])

    return norm_kernel


def _dinv_body_tc(degp_ref, o_ref):
    d = degp_ref[0] + degp_ref[1]
    pos = d > 0.0
    safe = jnp.where(pos, d, 1.0)
    o_ref[...] = jnp.where(pos, lax.rsqrt(safe), 0.0)


def _dinv_tc(degp):
    _, r, c = degp.shape
    return pl.pallas_call(
        _dinv_body_tc,
        out_shape=jax.ShapeDtypeStruct((r, c), jnp.float32),
    )(degp)


def _make_scatter(N, T, E, Cout, R, W, K):
    NT = N * T
    NW = 32
    NPASS = -(-N // (NW * R))
    NWIN = E // W
    WG = W // L
    KB = T * K
    GC = 256
    MB = -(-(W + L) // GC) * GC

    mesh = plsc.VectorSubcoreMesh(core_axis_name="c", subcore_axis_name="s")

    @functools.partial(
        pl.kernel,
        out_type=jax.ShapeDtypeStruct((NT, Cout), jnp.float32),
        mesh=mesh,
        compiler_params=pltpu.CompilerParams(needs_layout_passes=False),
        scratch_types=[
            pltpu.VMEM((T * R, Cout), jnp.float32),
            pltpu.VMEM((W,), jnp.int32),
            pltpu.VMEM((MB,), jnp.int32),
            pltpu.VMEM((MB,), jnp.int32),
            pltpu.VMEM((MB,), jnp.float32),
            pltpu.VMEM((KB, Cout), jnp.float32),
            pltpu.SemaphoreType.DMA,
            pltpu.SemaphoreType.DMA,
        ],
    )
    def scatter_kernel(row_hbm, col_hbm, norm_hbm, h_hbm, agg_hbm,
                       acc, win, mpos, rbufw, nbufw, staging, sem0, sem1):
        cid = lax.axis_index("c")
        sid = lax.axis_index("s")
        wid = cid * 16 + sid
        zero16 = jnp.zeros((L,), jnp.float32)
        iota16 = lax.iota(jnp.int32, L)

        def _pass_body(p, _):
            lo = pl.multiple_of(
                jnp.minimum((p * NW + wid) * R, N - R), 8)
            hi = lo + R

            @plsc.parallel_loop(0, T * R, unroll=2)
            def _zrow(r):
                for g in range(Cout // L):
                    acc[r, pl.ds(g * L, L)] = zero16

            def _win_body(w, _):
                pltpu.sync_copy(col_hbm.at[pl.ds(w * W, W)], win)

                @plsc.parallel_loop(0, WG, unroll=4, carry=jnp.int32(0))
                def _scan_body(g, cnt):
                    c16 = win[pl.ds(g * L, L)]
                    m = (c16 >= lo) & (c16 < hi)
                    plsc.store_compressed(mpos.at[pl.ds(cnt, L)],
                                          g * L + iota16, mask=m)
                    pc = plsc.all_reduce_population_count(m)
                    return cnt + pc[0]
                cnt = _scan_body

                mpos[pl.ds(cnt, L)] = jnp.zeros((L,), jnp.int32)
                nbatch = (cnt + K - 1) // K

                def _bulk(ch, _):
                    off = ch * GC
                    def _g2e(g, _):
                        mp = mpos[pl.ds(off + g * L, L)]
                        mpos[pl.ds(off + g * L, L)] = jnp.minimum(
                            mp, W - 1) + w * W
                        return 0
                    lax.fori_loop(0, GC // L, _g2e, 0)
                    cpa = pltpu.async_copy(
                        row_hbm.at[mpos.at[pl.ds(off, GC)]],
                        rbufw.at[pl.ds(off, GC)], sem0)
                    cpb = pltpu.async_copy(
                        norm_hbm.at[mpos.at[pl.ds(off, GC)]],
                        nbufw.at[pl.ds(off, GC)], sem1)
                    cpa.wait()
                    cpb.wait()
                    return 0
                nchunk = (cnt + L + GC - 1) // GC
                lax.fori_loop(0, nchunk, _bulk, 0)

                def _batch_body(b, _):
                    e0 = b * K
                    r16 = rbufw[pl.ds(e0, L)]
                    cps0 = []
                    cps1 = []
                    for t in range(T):
                        cp = pltpu.async_copy(
                            h_hbm.at[r16 + t * N],
                            staging.at[pl.ds(t * K, K)],
                            sem0 if t < T // 2 else sem1)
                        (cps0 if t < T // 2 else cps1).append(cp)

                    def _edge_range(t0, t1):
                        @plsc.parallel_loop(0, K)
                        def _edge(j):
                            mpv = plsc.load_gather(
                                mpos, [jnp.broadcast_to(e0 + j, (L,))])
                            lcolv = plsc.load_gather(
                                win, [jnp.minimum(mpv - w * W, W - 1)])
                            lcol = lcolv[0] - lo
                            valid = (e0 + j) < cnt
                            lc = jnp.clip(lcol, 0, R - 1)
                            nsv = plsc.load_gather(
                                nbufw, [jnp.broadcast_to(e0 + j, (L,))])
                            nsv = jnp.where(valid, nsv, zero16)
                            for t in range(t0, t1):
                                for g in range(Cout // L):
                                    v = staging[t * K + j,
                                                pl.ds(g * L, L)] * nsv
                                    plsc.addupdate(
                                        acc.at[t * R + lc, pl.ds(g * L, L)],
                                        v)
                    for cp in cps0:
                        cp.wait()
                    _edge_range(0, T // 2)
                    for cp in cps1:
                        cp.wait()
                    _edge_range(T // 2, T)
                    return 0
                lax.fori_loop(0, nbatch, _batch_body, 0)
                return 0
            lax.fori_loop(0, NWIN, _win_body, 0)

            for t in range(T):
                pltpu.sync_copy(acc.at[pl.ds(t * R, R)],
                                agg_hbm.at[pl.ds(t * N + lo, R)])
            return 0
        lax.fori_loop(0, NPASS, _pass_body, 0)

    return scatter_kernel


def kernel(x, edge_index, edge_attr, W_init, W_root, bias):
    N, T, C = x.shape
    E = edge_attr.shape[0]
    NT = N * T
    Cout = W_init.shape[1]
    x_flat = x.reshape(NT, C)
    row = edge_index[0]
    col = edge_index[1]

    NP = 10240
    deg = _make_deg(N, E, NP)(col, edge_attr)
    dinv = _dinv_tc(deg.reshape(2, NP // 128, 128)).reshape(NP)
    norm = _make_norm(N, E, NP)(row, col, edge_attr, dinv)

    h = _matmul(x_flat, W_init, bm=800)

    scatter = _make_scatter(N, T, E, Cout, R=32, W=6400, K=16)
    agg = scatter(row, col, norm, h)

    out = _epilogue(agg, x_flat, W_root, bias.reshape(1, Cout), bm=800)
    return out.reshape(N, T, Cout)
